# baseline (speedup 1.0000x reference)
import jax
import jax.numpy as jnp
from jax import lax
from jax.experimental import pallas as pl
from jax.experimental.pallas import tpu as pltpu
from jax.experimental.pallas import tpu_sc as plsc

MAXN = 100000
SEQL = 512
NC = 2
NS = 16
NW = NC * NS
SLAB = 3128
SLAB_LAST = MAXN - (NW - 1) * SLAB
SEG = 16


def _body(seq_in, len_in, lp_in, sc_in, nseq_in,
          seq_out, len_out, lp_out, size_out,
          sc_v, seg_i, seg_f, szo_v, blk_v):
    wid = lax.axis_index("s") * NC + lax.axis_index("c")

    pltpu.sync_copy(sc_in, sc_v)
    sc_vec = sc_v[...]
    s = sc_vec[0]
    nlen = sc_vec[1]
    nlp = lax.bitcast_convert_type(sc_vec[2], jnp.float32)

    r0 = wid * SLAB

    @pl.when(wid < NW - 1)
    def _():
        pltpu.sync_copy(seq_in.at[pl.ds(r0, SLAB)],
                        seq_out.at[pl.ds(r0, SLAB)])

    @pl.when(wid == NW - 1)
    def _():
        pltpu.sync_copy(seq_in.at[pl.ds((NW - 1) * SLAB, SLAB_LAST)],
                        seq_out.at[pl.ds((NW - 1) * SLAB, SLAB_LAST)])

    hi = jnp.where(wid == NW - 1, MAXN, r0 + SLAB)

    @pl.when((s >= r0) & (s < hi))
    def _():
        rb = (s // 8) * 8
        pltpu.sync_copy(seq_in.at[pl.ds(rb, 8)], blk_v)
        pltpu.sync_copy(nseq_in, blk_v.at[s - rb])
        pltpu.sync_copy(blk_v, seq_out.at[pl.ds(rb, 8)])

    lane = lax.iota(jnp.int32, SEG)
    b16 = (s // SEG) * SEG
    c = s - b16

    @pl.when(wid == 1)
    def _():
        pltpu.sync_copy(len_in, len_out)

        @pl.when(s < MAXN)
        def _():
            pltpu.sync_copy(len_in.at[pl.ds(b16, SEG)], seg_i)
            seg_i[...] = jnp.where(lane == c, nlen, seg_i[...])
            pltpu.sync_copy(seg_i, len_out.at[pl.ds(b16, SEG)])

    @pl.when(wid == 2)
    def _():
        pltpu.sync_copy(lp_in, lp_out)

        @pl.when(s < MAXN)
        def _():
            pltpu.sync_copy(lp_in.at[pl.ds(b16, SEG)], seg_f)
            seg_f[...] = jnp.where(lane == c, nlp, seg_f[...])
            pltpu.sync_copy(seg_f, lp_out.at[pl.ds(b16, SEG)])

    @pl.when(wid == 3)
    def _():
        szo_v[...] = jnp.where(lane == 0, s + 1, 0)
        pltpu.sync_copy(szo_v, size_out)


_tree_add = pl.kernel(
    _body,
    out_type=(
        jax.ShapeDtypeStruct((MAXN, SEQL), jnp.int32),
        jax.ShapeDtypeStruct((MAXN,), jnp.int32),
        jax.ShapeDtypeStruct((MAXN,), jnp.float32),
        jax.ShapeDtypeStruct((SEG,), jnp.int32),
    ),
    mesh=plsc.VectorSubcoreMesh(core_axis_name="c", subcore_axis_name="s"),
    scratch_types=[
        pltpu.VMEM((SEG,), jnp.int32),
        pltpu.VMEM((SEG,), jnp.int32),
        pltpu.VMEM((SEG,), jnp.float32),
        pltpu.VMEM((SEG,), jnp.int32),
        pltpu.VMEM((8, SEQL), jnp.int32),
    ],
)


def kernel(sequences, sequence_lengths, log_probabilities, size,
           node_sequence, node_sequence_length, node_log_probability):
    lp_bits = lax.bitcast_convert_type(node_log_probability, jnp.int32)
    scalars = (jnp.zeros((SEG,), jnp.int32)
               .at[0].set(size)
               .at[1].set(node_sequence_length)
               .at[2].set(lp_bits))
    seq_o, len_o, lp_o, size_o = _tree_add(
        sequences, sequence_lengths, log_probabilities, scalars,
        node_sequence)
    return (seq_o, len_o, lp_o, size_o[0])


# speedup vs baseline: 29.2805x; 29.2805x over previous
import jax
import jax.numpy as jnp
from jax import lax
from jax.experimental import pallas as pl
from jax.experimental.pallas import tpu as pltpu
from jax.experimental.pallas import tpu_sc as plsc

MAXN = 100000
SEQL = 512
NC = 2
NS = 16
NW = NC * NS
CH_ROWS = 80
NCHUNKS = MAXN // CH_ROWS
NPW = -(-NCHUNKS // NW)
SEG = 16


def _body(seq_in, len_in, lp_in, sc_in, nseq_in,
          seq_out, len_out, lp_out, size_out,
          sc_v, seg_i, seg_f, szo_v, blk_v, buf_v):
    wid = lax.axis_index("s") * NC + lax.axis_index("c")

    pltpu.sync_copy(sc_in, sc_v)
    sc_vec = sc_v[...]
    s = sc_vec[0]
    nlen = sc_vec[1]
    nlp = lax.bitcast_convert_type(sc_vec[2], jnp.float32)

    base = wid * NPW
    for j in range(NPW):
        cid = base + j
        r = cid * CH_ROWS

        @pl.when(cid < NCHUNKS)
        def _():
            pltpu.sync_copy(seq_in.at[pl.ds(r, CH_ROWS)], buf_v)
            pltpu.sync_copy(buf_v, seq_out.at[pl.ds(r, CH_ROWS)])

    cs = s // CH_ROWS

    @pl.when((cs >= base) & (cs < base + NPW))
    def _():
        rb = (s // 8) * 8
        pltpu.sync_copy(seq_in.at[pl.ds(rb, 8)], blk_v)
        pltpu.sync_copy(nseq_in, blk_v.at[s - rb])
        pltpu.sync_copy(blk_v, seq_out.at[pl.ds(rb, 8)])

    lane = lax.iota(jnp.int32, SEG)
    b16 = (s // SEG) * SEG
    c = s - b16

    @pl.when(wid == 1)
    def _():
        pltpu.sync_copy(len_in, len_out)

        @pl.when(s < MAXN)
        def _():
            pltpu.sync_copy(len_in.at[pl.ds(b16, SEG)], seg_i)
            seg_i[...] = jnp.where(lane == c, nlen, seg_i[...])
            pltpu.sync_copy(seg_i, len_out.at[pl.ds(b16, SEG)])

    @pl.when(wid == 2)
    def _():
        pltpu.sync_copy(lp_in, lp_out)

        @pl.when(s < MAXN)
        def _():
            pltpu.sync_copy(lp_in.at[pl.ds(b16, SEG)], seg_f)
            seg_f[...] = jnp.where(lane == c, nlp, seg_f[...])
            pltpu.sync_copy(seg_f, lp_out.at[pl.ds(b16, SEG)])

    @pl.when(wid == 3)
    def _():
        szo_v[...] = jnp.where(lane == 0, s + 1, 0)
        pltpu.sync_copy(szo_v, size_out)


_tree_add = pl.kernel(
    _body,
    out_type=(
        jax.ShapeDtypeStruct((MAXN, SEQL), jnp.int32),
        jax.ShapeDtypeStruct((MAXN,), jnp.int32),
        jax.ShapeDtypeStruct((MAXN,), jnp.float32),
        jax.ShapeDtypeStruct((SEG,), jnp.int32),
    ),
    mesh=plsc.VectorSubcoreMesh(core_axis_name="c", subcore_axis_name="s"),
    scratch_types=[
        pltpu.VMEM((SEG,), jnp.int32),
        pltpu.VMEM((SEG,), jnp.int32),
        pltpu.VMEM((SEG,), jnp.float32),
        pltpu.VMEM((SEG,), jnp.int32),
        pltpu.VMEM((8, SEQL), jnp.int32),
        pltpu.VMEM((CH_ROWS, SEQL), jnp.int32),
    ],
)


def kernel(sequences, sequence_lengths, log_probabilities, size,
           node_sequence, node_sequence_length, node_log_probability):
    lp_bits = lax.bitcast_convert_type(node_log_probability, jnp.int32)
    scalars = (jnp.zeros((SEG,), jnp.int32)
               .at[0].set(size)
               .at[1].set(node_sequence_length)
               .at[2].set(lp_bits))
    seq_o, len_o, lp_o, size_o = _tree_add(
        sequences, sequence_lengths, log_probabilities, scalars,
        node_sequence)
    return (seq_o, len_o, lp_o, size_o[0])


# speedup vs baseline: 36.9991x; 1.2636x over previous
import jax
import jax.numpy as jnp
from jax import lax
from jax.experimental import pallas as pl
from jax.experimental.pallas import tpu as pltpu
from jax.experimental.pallas import tpu_sc as plsc

MAXN = 100000
SEQL = 512
NC = 2
NS = 16
NW = NC * NS
CH_ROWS = 80
NCHUNKS = MAXN // CH_ROWS
NPW = -(-NCHUNKS // NW)
SEG = 16
PIECE = 5000


def _body(seq_in, len_in, lp_in, sc_in, nseq_in,
          seq_out, len_out, lp_out, size_out,
          sc_v, seg_i, seg_f, szo_v, blk_v, buf0, buf1, pc_i, pc_f,
          gsem0, gsem1, ssem0, ssem1):
    wid = lax.axis_index("s") * NC + lax.axis_index("c")
    bufs = (buf0, buf1)
    gsems = (gsem0, gsem1)
    ssems = (ssem0, ssem1)

    pltpu.sync_copy(sc_in, sc_v)
    sc_vec = sc_v[...]
    s = sc_vec[0]
    nlen = sc_vec[1]
    nlp = lax.bitcast_convert_type(sc_vec[2], jnp.float32)

    base = wid * NPW
    sd = [None] * NPW
    for j in range(NPW):
        b = j % 2
        cid = base + j
        r = cid * CH_ROWS

        if j >= 2:
            @pl.when(base + j - 2 < NCHUNKS)
            def _():
                sd[j - 2].wait()

        @pl.when(cid < NCHUNKS)
        def _():
            gd = pltpu.async_copy(seq_in.at[pl.ds(r, CH_ROWS)], bufs[b],
                                  gsems[b])
            gd.wait()
            sd[j] = pltpu.async_copy(bufs[b], seq_out.at[pl.ds(r, CH_ROWS)],
                                     ssems[b])

    for j in range(max(0, NPW - 2), NPW):
        @pl.when(base + j < NCHUNKS)
        def _():
            sd[j].wait()

    cs = s // CH_ROWS

    @pl.when((cs >= base) & (cs < base + NPW))
    def _():
        rb = (s // 8) * 8
        pltpu.sync_copy(seq_in.at[pl.ds(rb, 8)], blk_v)
        pltpu.sync_copy(nseq_in, blk_v.at[s - rb])
        pltpu.sync_copy(blk_v, seq_out.at[pl.ds(rb, 8)])

    lane = lax.iota(jnp.int32, SEG)
    b16 = (s // SEG) * SEG
    c = s - b16

    @pl.when(wid == NW - 1)
    def _():
        for t in range(MAXN // PIECE):
            pltpu.sync_copy(len_in.at[pl.ds(t * PIECE, PIECE)], pc_i)
            pltpu.sync_copy(pc_i, len_out.at[pl.ds(t * PIECE, PIECE)])
            pltpu.sync_copy(lp_in.at[pl.ds(t * PIECE, PIECE)], pc_f)
            pltpu.sync_copy(pc_f, lp_out.at[pl.ds(t * PIECE, PIECE)])

        @pl.when(s < MAXN)
        def _():
            pltpu.sync_copy(len_in.at[pl.ds(b16, SEG)], seg_i)
            seg_i[...] = jnp.where(lane == c, nlen, seg_i[...])
            pltpu.sync_copy(seg_i, len_out.at[pl.ds(b16, SEG)])
            pltpu.sync_copy(lp_in.at[pl.ds(b16, SEG)], seg_f)
            seg_f[...] = jnp.where(lane == c, nlp, seg_f[...])
            pltpu.sync_copy(seg_f, lp_out.at[pl.ds(b16, SEG)])

        szo_v[...] = jnp.where(lane == 0, s + 1, 0)
        pltpu.sync_copy(szo_v, size_out)


_tree_add = pl.kernel(
    _body,
    out_type=(
        jax.ShapeDtypeStruct((MAXN, SEQL), jnp.int32),
        jax.ShapeDtypeStruct((MAXN,), jnp.int32),
        jax.ShapeDtypeStruct((MAXN,), jnp.float32),
        jax.ShapeDtypeStruct((SEG,), jnp.int32),
    ),
    mesh=plsc.VectorSubcoreMesh(core_axis_name="c", subcore_axis_name="s"),
    scratch_types=[
        pltpu.VMEM((SEG,), jnp.int32),
        pltpu.VMEM((SEG,), jnp.int32),
        pltpu.VMEM((SEG,), jnp.float32),
        pltpu.VMEM((SEG,), jnp.int32),
        pltpu.VMEM((8, SEQL), jnp.int32),
        pltpu.VMEM((CH_ROWS, SEQL), jnp.int32),
        pltpu.VMEM((CH_ROWS, SEQL), jnp.int32),
        pltpu.VMEM((PIECE,), jnp.int32),
        pltpu.VMEM((PIECE,), jnp.float32),
        pltpu.SemaphoreType.DMA,
        pltpu.SemaphoreType.DMA,
        pltpu.SemaphoreType.DMA,
        pltpu.SemaphoreType.DMA,
    ],
)


def kernel(sequences, sequence_lengths, log_probabilities, size,
           node_sequence, node_sequence_length, node_log_probability):
    lp_bits = lax.bitcast_convert_type(node_log_probability, jnp.int32)
    scalars = (jnp.zeros((SEG,), jnp.int32)
               .at[0].set(size)
               .at[1].set(node_sequence_length)
               .at[2].set(lp_bits))
    seq_o, len_o, lp_o, size_o = _tree_add(
        sequences, sequence_lengths, log_probabilities, scalars,
        node_sequence)
    return (seq_o, len_o, lp_o, size_o[0])
